# baseline (speedup 1.0000x reference)
import functools

import jax
import jax.numpy as jnp
from jax.experimental import pallas as pl
from jax.experimental.pallas import tpu as pltpu

EPS = 1e-12
BN = 2000
BN2 = 2000

_DN0 = (((0,), (0,)), ((), ()))


def _pass_in(adj_ref, x_ref, t_ref, col_ref, adjb_ref, ones_ref):
    i = pl.program_id(0)

    @pl.when(i == 0)
    def _init():
        t_ref[...] = jnp.zeros_like(t_ref)
        col_ref[...] = jnp.zeros_like(col_ref)
        ones_ref[...] = jnp.ones_like(ones_ref)

    adj = adj_ref[...]
    adjb_ref[...] = adj.astype(jnp.bfloat16)
    t_ref[...] += jax.lax.dot_general(
        x_ref[...], adj, _DN0, preferred_element_type=jnp.float32)
    col_ref[...] += jax.lax.dot_general(
        ones_ref[...], adj, _DN0, preferred_element_type=jnp.float32)


def _mkv(t, col_ref, w):
    col = jnp.maximum(col_ref[0:1, :], EPS)
    u = t / col
    return jax.lax.dot_general(
        u, w, _DN0, preferred_element_type=jnp.float32
    ).astype(jnp.bfloat16)


def _pass_fused(nblk, t0_ref, col_ref, ws_ref, bs_ref, adj_ref, out_ref,
                v_ref, ones_ref, t1_ref, t2_ref):
    i = pl.program_id(0)
    ph = i // nblk
    blk = i - ph * nblk

    @pl.when(i == 0)
    def _init0():
        ones_ref[...] = jnp.ones_like(ones_ref)
        t1_ref[...] = jnp.zeros_like(t1_ref)
        t2_ref[...] = jnp.zeros_like(t2_ref)
        v_ref[...] = _mkv(t0_ref[...], col_ref, ws_ref[0])

    @pl.when(i == nblk)
    def _init1():
        v_ref[...] = _mkv(t1_ref[...], col_ref, ws_ref[1])

    @pl.when(i == 2 * nblk)
    def _init2():
        v_ref[...] = _mkv(t2_ref[...], col_ref, ws_ref[2])

    adj = adj_ref[pl.ds(blk * BN2, BN2), :]
    row = jnp.maximum(
        jnp.dot(adj, ones_ref[...],
                preferred_element_type=jnp.float32)[:, 0:1], EPS)
    rinv = 1.0 / row
    y = jnp.dot(adj, v_ref[...], preferred_element_type=jnp.float32)
    z = y * rinv + bs_ref[pl.ds(ph, 1), :]

    @pl.when(ph == 0)
    def _acc1():
        g = jnp.maximum(z, 0.0).astype(jnp.bfloat16)
        t1_ref[...] += jax.lax.dot_general(
            g, adj, _DN0, preferred_element_type=jnp.float32)

    @pl.when(ph == 1)
    def _acc2():
        g = jnp.maximum(z, 0.0).astype(jnp.bfloat16)
        t2_ref[...] += jax.lax.dot_general(
            g, adj, _DN0, preferred_element_type=jnp.float32)

    @pl.when(ph == 2)
    def _emit():
        out_ref[...] = z[:, :out_ref.shape[1]]


@jax.jit
def kernel(x, node_anchor_adj, W1, b1, W2, b2, W3, b3):
    n, nfeat = x.shape
    a = node_anchor_adj.shape[1]
    nblk = n // BN
    adj = node_anchor_adj

    t0, col, adjb = pl.pallas_call(
        _pass_in,
        grid=(nblk,),
        in_specs=[
            pl.BlockSpec((BN, a), lambda i: (i, 0)),
            pl.BlockSpec((BN, nfeat), lambda i: (i, 0)),
        ],
        out_specs=[
            pl.BlockSpec((nfeat, a), lambda i: (0, 0)),
            pl.BlockSpec((8, a), lambda i: (0, 0)),
            pl.BlockSpec((BN, a), lambda i: (i, 0)),
        ],
        out_shape=[
            jax.ShapeDtypeStruct((nfeat, a), jnp.float32),
            jax.ShapeDtypeStruct((8, a), jnp.float32),
            jax.ShapeDtypeStruct((n, a), jnp.bfloat16),
        ],
        scratch_shapes=[pltpu.VMEM((BN, 8), jnp.float32)],
    )(adj, x)

    nh = W1.shape[1]
    nclass = W3.shape[1]
    ws = jnp.stack([W1, W2,
                    jnp.pad(W3, ((0, 0), (0, nh - nclass)))])
    bs = jnp.stack([b1, b2, jnp.pad(b3, (0, nh - nclass))])

    nblk2 = n // BN2
    out = pl.pallas_call(
        functools.partial(_pass_fused, nblk2),
        grid=(3 * nblk2,),
        in_specs=[
            pl.BlockSpec((nh, a), lambda i: (0, 0)),
            pl.BlockSpec((8, a), lambda i: (0, 0)),
            pl.BlockSpec((3, nh, nh), lambda i: (0, 0, 0)),
            pl.BlockSpec((3, nh), lambda i: (0, 0)),
            pl.BlockSpec((n, a), lambda i: (0, 0)),
        ],
        out_specs=pl.BlockSpec(
            (BN2, nclass),
            lambda i: (jnp.maximum(i - 2 * (n // BN2), 0), 0)),
        out_shape=jax.ShapeDtypeStruct((n, nclass), jnp.float32),
        scratch_shapes=[
            pltpu.VMEM((a, nh), jnp.bfloat16),
            pltpu.VMEM((a, 8), jnp.bfloat16),
            pltpu.VMEM((nh, a), jnp.float32),
            pltpu.VMEM((nh, a), jnp.float32),
        ],
    )(t0, col, ws, bs, adjb)
    return out


# speedup vs baseline: 1.4685x; 1.4685x over previous
import functools

import jax
import jax.numpy as jnp
from jax.experimental import pallas as pl
from jax.experimental.pallas import tpu as pltpu

EPS = 1e-12
BN = 2000
BN2 = 10000
NSUB = 5

_DN0 = (((0,), (0,)), ((), ()))


def _pass_in(adj_ref, x_ref, t_ref, col_ref, adjb_ref, ones_ref):
    i = pl.program_id(0)

    @pl.when(i == 0)
    def _init():
        t_ref[...] = jnp.zeros_like(t_ref)
        col_ref[...] = jnp.zeros_like(col_ref)
        ones_ref[...] = jnp.ones_like(ones_ref)

    adj = adj_ref[...]
    adjb_ref[...] = adj.astype(jnp.bfloat16)
    t_ref[...] += jax.lax.dot_general(
        x_ref[...], adj, _DN0, preferred_element_type=jnp.float32)
    col_ref[...] += jax.lax.dot_general(
        ones_ref[...], adj, _DN0, preferred_element_type=jnp.float32)


def _mkv(t, col_ref, w):
    col = jnp.maximum(col_ref[0:1, :], EPS)
    u = t / col
    v = jax.lax.dot_general(
        u, w, _DN0, preferred_element_type=jnp.float32)
    ones = jnp.ones((v.shape[0], 256 - v.shape[1]), jnp.float32)
    return jnp.concatenate([v, ones], axis=1).astype(jnp.bfloat16)


def _pass_fused(nblk, t0_ref, col_ref, ws_ref, bs_ref, adj_ref, out_ref,
                v_ref, t1_ref, t2_ref):
    i = pl.program_id(0)
    ph = i // nblk

    @pl.when(i == 0)
    def _init0():
        t1_ref[...] = jnp.zeros_like(t1_ref)
        t2_ref[...] = jnp.zeros_like(t2_ref)
        v_ref[...] = _mkv(t0_ref[...], col_ref, ws_ref[0])

    @pl.when(i == nblk)
    def _init1():
        v_ref[...] = _mkv(t1_ref[...], col_ref, ws_ref[1])

    @pl.when(i == 2 * nblk)
    def _init2():
        v_ref[...] = _mkv(t2_ref[...], col_ref, ws_ref[2])

    bvec = bs_ref[pl.ds(ph, 1), :]
    nh = bvec.shape[1]
    sub = BN2 // NSUB
    tiles = []
    for h in range(NSUB):
        adj_h = adj_ref[h * sub:(h + 1) * sub, :]
        ya = jnp.dot(adj_h, v_ref[...], preferred_element_type=jnp.float32)
        row = jnp.maximum(ya[:, nh:nh + 1], EPS)
        z = ya[:, :nh] * (1.0 / row) + bvec
        tiles.append((adj_h, z))

    def _acc(t_ref):
        acc = None
        for adj_h, z in tiles:
            g = jnp.maximum(z, 0.0).astype(jnp.bfloat16)
            d = jax.lax.dot_general(
                g, adj_h, _DN0, preferred_element_type=jnp.float32)
            acc = d if acc is None else acc + d
        t_ref[...] += acc

    @pl.when(ph == 0)
    def _acc1():
        _acc(t1_ref)

    @pl.when(ph == 1)
    def _acc2():
        _acc(t2_ref)

    @pl.when(ph == 2)
    def _emit():
        nc = out_ref.shape[1]
        for h, (adj_h, z) in enumerate(tiles):
            out_ref[h * sub:(h + 1) * sub, :] = z[:, :nc]


@jax.jit
def kernel(x, node_anchor_adj, W1, b1, W2, b2, W3, b3):
    n, nfeat = x.shape
    a = node_anchor_adj.shape[1]
    nblk = n // BN
    adj = node_anchor_adj

    t0, col, adjb = pl.pallas_call(
        _pass_in,
        grid=(nblk,),
        in_specs=[
            pl.BlockSpec((BN, a), lambda i: (i, 0)),
            pl.BlockSpec((BN, nfeat), lambda i: (i, 0)),
        ],
        out_specs=[
            pl.BlockSpec((nfeat, a), lambda i: (0, 0)),
            pl.BlockSpec((8, a), lambda i: (0, 0)),
            pl.BlockSpec((BN, a), lambda i: (i, 0)),
        ],
        out_shape=[
            jax.ShapeDtypeStruct((nfeat, a), jnp.float32),
            jax.ShapeDtypeStruct((8, a), jnp.float32),
            jax.ShapeDtypeStruct((n, a), jnp.bfloat16),
        ],
        scratch_shapes=[pltpu.VMEM((BN, 8), jnp.float32)],
    )(adj, x)

    nh = W1.shape[1]
    nclass = W3.shape[1]
    ws = jnp.stack([W1, W2,
                    jnp.pad(W3, ((0, 0), (0, nh - nclass)))])
    bs = jnp.stack([b1, b2, jnp.pad(b3, (0, nh - nclass))])

    nblk2 = n // BN2
    out = pl.pallas_call(
        functools.partial(_pass_fused, nblk2),
        grid=(3 * nblk2,),
        in_specs=[
            pl.BlockSpec((nh, a), lambda i: (0, 0)),
            pl.BlockSpec((8, a), lambda i: (0, 0)),
            pl.BlockSpec((3, nh, nh), lambda i: (0, 0, 0)),
            pl.BlockSpec((3, nh), lambda i: (0, 0)),
            pl.BlockSpec((BN2, a), lambda i: (i % (n // BN2), 0)),
        ],
        out_specs=pl.BlockSpec(
            (BN2, nclass),
            lambda i: (jnp.maximum(i - 2 * (n // BN2), 0), 0)),
        out_shape=jax.ShapeDtypeStruct((n, nclass), jnp.float32),
        scratch_shapes=[
            pltpu.VMEM((a, 256), jnp.bfloat16),
            pltpu.VMEM((nh, a), jnp.float32),
            pltpu.VMEM((nh, a), jnp.float32),
        ],
    )(t0, col, ws, bs, adjb)
    return out


# speedup vs baseline: 1.5193x; 1.0346x over previous
import functools

import jax
import jax.numpy as jnp
from jax.experimental import pallas as pl
from jax.experimental.pallas import tpu as pltpu

EPS = 1e-12
BN = 5000
BN2 = 10000
NSUB = 5

_DN0 = (((0,), (0,)), ((), ()))


def _pass_in(adj_ref, x_ref, t_ref, col_ref, adjb_ref, ones_ref):
    i = pl.program_id(0)

    @pl.when(i == 0)
    def _init():
        t_ref[...] = jnp.zeros_like(t_ref)
        col_ref[...] = jnp.zeros_like(col_ref)
        ones_ref[...] = jnp.ones_like(ones_ref)

    adj = adj_ref[...]
    adjb_ref[...] = adj.astype(jnp.bfloat16)
    t_ref[...] += jax.lax.dot_general(
        x_ref[...], adj, _DN0, preferred_element_type=jnp.float32)
    col_ref[...] += jax.lax.dot_general(
        ones_ref[...], adj, _DN0, preferred_element_type=jnp.float32)


def _mkv(t, col_ref, w):
    col = jnp.maximum(col_ref[0:1, :], EPS)
    u = t / col
    v = jax.lax.dot_general(
        u, w, _DN0, preferred_element_type=jnp.float32)
    ones = jnp.ones((v.shape[0], 256 - v.shape[1]), jnp.float32)
    return jnp.concatenate([v, ones], axis=1).astype(jnp.bfloat16)


def _pass_fused(nblk, t0_ref, col_ref, ws_ref, bs_ref, adj_ref, out_ref,
                v_ref, t1_ref, t2_ref):
    i = pl.program_id(0)
    ph = i // nblk

    @pl.when(i == 0)
    def _init0():
        t1_ref[...] = jnp.zeros_like(t1_ref)
        t2_ref[...] = jnp.zeros_like(t2_ref)
        v_ref[...] = _mkv(t0_ref[...], col_ref, ws_ref[0])

    @pl.when(i == nblk)
    def _init1():
        v_ref[...] = _mkv(t1_ref[...], col_ref, ws_ref[1])

    @pl.when(i == 2 * nblk)
    def _init2():
        v_ref[...] = _mkv(t2_ref[...], col_ref, ws_ref[2])

    bvec = bs_ref[pl.ds(ph, 1), :]
    nh = bvec.shape[1]
    sub = BN2 // NSUB
    tiles = []
    for h in range(NSUB):
        adj_h = adj_ref[h * sub:(h + 1) * sub, :]
        ya = jnp.dot(adj_h, v_ref[...], preferred_element_type=jnp.float32)
        row = jnp.maximum(ya[:, nh:nh + 1], EPS)
        z = ya[:, :nh] * (1.0 / row) + bvec
        tiles.append((adj_h, z))

    def _acc(t_ref):
        acc = None
        for adj_h, z in tiles:
            g = jnp.maximum(z, 0.0).astype(jnp.bfloat16)
            d = jax.lax.dot_general(
                g, adj_h, _DN0, preferred_element_type=jnp.float32)
            acc = d if acc is None else acc + d
        t_ref[...] += acc

    @pl.when(ph == 0)
    def _acc1():
        _acc(t1_ref)

    @pl.when(ph == 1)
    def _acc2():
        _acc(t2_ref)

    @pl.when(ph == 2)
    def _emit():
        nc = out_ref.shape[1]
        for h, (adj_h, z) in enumerate(tiles):
            out_ref[h * sub:(h + 1) * sub, :] = z[:, :nc]


@jax.jit
def kernel(x, node_anchor_adj, W1, b1, W2, b2, W3, b3):
    n, nfeat = x.shape
    a = node_anchor_adj.shape[1]
    nblk = n // BN
    adj = node_anchor_adj

    t0, col, adjb = pl.pallas_call(
        _pass_in,
        grid=(nblk,),
        in_specs=[
            pl.BlockSpec((BN, a), lambda i: (i, 0)),
            pl.BlockSpec((BN, nfeat), lambda i: (i, 0)),
        ],
        out_specs=[
            pl.BlockSpec((nfeat, a), lambda i: (0, 0)),
            pl.BlockSpec((8, a), lambda i: (0, 0)),
            pl.BlockSpec((BN, a), lambda i: (i, 0)),
        ],
        out_shape=[
            jax.ShapeDtypeStruct((nfeat, a), jnp.float32),
            jax.ShapeDtypeStruct((8, a), jnp.float32),
            jax.ShapeDtypeStruct((n, a), jnp.bfloat16),
        ],
        scratch_shapes=[pltpu.VMEM((BN, 8), jnp.float32)],
    )(adj, x)

    nh = W1.shape[1]
    nclass = W3.shape[1]
    ws = jnp.stack([W1, W2,
                    jnp.pad(W3, ((0, 0), (0, nh - nclass)))])
    bs = jnp.stack([b1, b2, jnp.pad(b3, (0, nh - nclass))])

    nblk2 = n // BN2
    out = pl.pallas_call(
        functools.partial(_pass_fused, nblk2),
        grid=(3 * nblk2,),
        in_specs=[
            pl.BlockSpec((nh, a), lambda i: (0, 0)),
            pl.BlockSpec((8, a), lambda i: (0, 0)),
            pl.BlockSpec((3, nh, nh), lambda i: (0, 0, 0)),
            pl.BlockSpec((3, nh), lambda i: (0, 0)),
            pl.BlockSpec((BN2, a), lambda i: (i % (n // BN2), 0)),
        ],
        out_specs=pl.BlockSpec(
            (BN2, nclass),
            lambda i: (jnp.maximum(i - 2 * (n // BN2), 0), 0)),
        out_shape=jax.ShapeDtypeStruct((n, nclass), jnp.float32),
        scratch_shapes=[
            pltpu.VMEM((a, 256), jnp.bfloat16),
            pltpu.VMEM((nh, a), jnp.float32),
            pltpu.VMEM((nh, a), jnp.float32),
        ],
    )(t0, col, ws, bs, adjb)
    return out
